# baseline (speedup 1.0000x reference)
import functools

import jax
import jax.numpy as jnp
from jax import lax
from jax.experimental import pallas as pl
from jax.experimental.pallas import tpu as pltpu
from jax.experimental.pallas import tpu_sc as plsc
import numpy as np

_FEATURE_FIELDS = [100000] * 26
_F = len(_FEATURE_FIELDS)
_D = 16
_B = 16384
_H = 400
_EIN = _F * _D
_OFFS = np.array((0, *np.cumsum(_FEATURE_FIELDS)[:-1]), dtype=np.int32)

_NW = 32
_BPW = _B // _NW
_CHUNK = 128
_NCH = _BPW // _CHUNK
_IPC = _CHUNK * _F
_IROWS = _F


def _sc_gather_body(tmp_hbm, emb_hbm, lin_hbm, emb_out, lin_out,
                    idx_v, emb_v, lin_v, sem_e, sem_l):
    nc = 2
    wid = lax.axis_index("s") * nc + lax.axis_index("c")

    def round_body(ci, _):
        base = wid * _NCH + ci
        r0 = base * _IPC
        pltpu.sync_copy(tmp_hbm.at[base], idx_v)
        cps = []
        for j in range(_IROWS):
            cps.append(pltpu.async_copy(
                emb_hbm.at[idx_v.at[j]],
                emb_v.at[pl.ds(j * _CHUNK, _CHUNK)], sem_e))
            cps.append(pltpu.async_copy(
                lin_hbm.at[idx_v.at[j]],
                lin_v.at[pl.ds(j * _CHUNK, _CHUNK)], sem_l))
        for cp in cps:
            cp.wait()
        pltpu.sync_copy(emb_v, emb_out.at[pl.ds(r0, _IPC)])
        pltpu.sync_copy(lin_v, lin_out.at[pl.ds(r0, _IPC)])
        return _

    lax.fori_loop(0, _NCH, round_body, 0)


def _sc_gather(tmp3d, emb_table, lin_table):
    k = functools.partial(
        pl.kernel,
        out_type=[
            jax.ShapeDtypeStruct((_B * _F, _D), jnp.float32),
            jax.ShapeDtypeStruct((_B * _F,), jnp.float32),
        ],
        mesh=plsc.VectorSubcoreMesh(core_axis_name="c", subcore_axis_name="s"),
        compiler_params=pltpu.CompilerParams(use_tc_tiling_on_sc=False),
        scratch_types=[
            pltpu.VMEM((_IROWS, _CHUNK), jnp.int32),
            pltpu.VMEM((_IPC, _D), jnp.float32),
            pltpu.VMEM((_IPC,), jnp.float32),
            pltpu.SemaphoreType.DMA,
            pltpu.SemaphoreType.DMA,
        ],
    )(_sc_gather_body)
    return k(tmp3d, emb_table, lin_table)


_TB = 512


def _passA_kernel(e_ref, lv_ref, w1_ref, b1_ref, bias_ref,
                  h1_ref, s_ref, q_ref, fm_ref):
    @pl.when(pl.program_id(0) == 0)
    def _():
        s_ref[...] = jnp.zeros_like(s_ref)
        q_ref[...] = jnp.zeros_like(q_ref)

    e = e_ref[...]
    h1 = jnp.dot(e, w1_ref[...], preferred_element_type=jnp.float32)
    h1 = h1 + b1_ref[...]
    h1_ref[...] = h1
    s_ref[...] += jnp.sum(h1, axis=0, keepdims=True)
    q_ref[...] += jnp.sum(h1 * h1, axis=0, keepdims=True)

    es = jnp.zeros((_TB, _D), jnp.float32)
    qs = jnp.zeros((_TB, _D), jnp.float32)
    for f in range(_F):
        v = e[:, f * _D:(f + 1) * _D]
        es = es + v
        qs = qs + v * v
    inner = 0.5 * jnp.sum(es * es - qs, axis=1, keepdims=True)
    linear = jnp.sum(lv_ref[...], axis=1, keepdims=True)
    fm_ref[...] = inner + linear + bias_ref[...]


def _passB_kernel(h1_ref, a1_ref, c1_ref, w2_ref, b2_ref,
                  h2_ref, s_ref, q_ref):
    @pl.when(pl.program_id(0) == 0)
    def _():
        s_ref[...] = jnp.zeros_like(s_ref)
        q_ref[...] = jnp.zeros_like(q_ref)

    h = jnp.maximum(h1_ref[...] * a1_ref[...] + c1_ref[...], 0.0)
    h2 = jnp.dot(h, w2_ref[...], preferred_element_type=jnp.float32)
    h2 = h2 + b2_ref[...]
    h2_ref[...] = h2
    s_ref[...] += jnp.sum(h2, axis=0, keepdims=True)
    q_ref[...] += jnp.sum(h2 * h2, axis=0, keepdims=True)


def _passC_kernel(h2_ref, a2_ref, c2_ref, w3_ref, b3_ref, fm_ref, out_ref):
    h = jnp.maximum(h2_ref[...] * a2_ref[...] + c2_ref[...], 0.0)
    mlp = jnp.dot(h, w3_ref[...], preferred_element_type=jnp.float32)
    out_ref[...] = jax.nn.sigmoid(mlp + b3_ref[...] + fm_ref[...])


def _bn_coeffs(s, q, g, be, eps=1e-5):
    m = s / _B
    v = q / _B - m * m
    a = g[None, :] * lax.rsqrt(v + eps)
    c = be[None, :] - m * a
    return a, c


def kernel(x, emb_table, lin_table, bias, W1, b1, g1, be1,
           W2, b2, g2, be2, W3, b3):
    tmp = x + jnp.asarray(_OFFS, dtype=x.dtype)[None, :]
    tmp3d = tmp.reshape(_B * _F // (_IROWS * _CHUNK), _IROWS, _CHUNK)

    emb_rows, lin_rows = _sc_gather(tmp3d, emb_table, lin_table[:, 0])
    e = emb_rows.reshape(_B, _EIN)
    lv = lin_rows.reshape(_B, _F)

    grid = (_B // _TB,)
    h1, s1, q1, fm = pl.pallas_call(
        _passA_kernel,
        grid=grid,
        in_specs=[
            pl.BlockSpec((_TB, _EIN), lambda t: (t, 0)),
            pl.BlockSpec((_TB, _F), lambda t: (t, 0)),
            pl.BlockSpec((_EIN, _H), lambda t: (0, 0)),
            pl.BlockSpec((1, _H), lambda t: (0, 0)),
            pl.BlockSpec((1, 1), lambda t: (0, 0)),
        ],
        out_specs=[
            pl.BlockSpec((_TB, _H), lambda t: (t, 0)),
            pl.BlockSpec((1, _H), lambda t: (0, 0)),
            pl.BlockSpec((1, _H), lambda t: (0, 0)),
            pl.BlockSpec((_TB, 1), lambda t: (t, 0)),
        ],
        out_shape=[
            jax.ShapeDtypeStruct((_B, _H), jnp.float32),
            jax.ShapeDtypeStruct((1, _H), jnp.float32),
            jax.ShapeDtypeStruct((1, _H), jnp.float32),
            jax.ShapeDtypeStruct((_B, 1), jnp.float32),
        ],
    )(e, lv, W1, b1[None, :], bias[None, :])

    a1, c1 = _bn_coeffs(s1[0], q1[0], g1, be1)

    h2, s2, q2 = pl.pallas_call(
        _passB_kernel,
        grid=grid,
        in_specs=[
            pl.BlockSpec((_TB, _H), lambda t: (t, 0)),
            pl.BlockSpec((1, _H), lambda t: (0, 0)),
            pl.BlockSpec((1, _H), lambda t: (0, 0)),
            pl.BlockSpec((_H, _H), lambda t: (0, 0)),
            pl.BlockSpec((1, _H), lambda t: (0, 0)),
        ],
        out_specs=[
            pl.BlockSpec((_TB, _H), lambda t: (t, 0)),
            pl.BlockSpec((1, _H), lambda t: (0, 0)),
            pl.BlockSpec((1, _H), lambda t: (0, 0)),
        ],
        out_shape=[
            jax.ShapeDtypeStruct((_B, _H), jnp.float32),
            jax.ShapeDtypeStruct((1, _H), jnp.float32),
            jax.ShapeDtypeStruct((1, _H), jnp.float32),
        ],
    )(h1, a1, c1, W2, b2[None, :])

    a2, c2 = _bn_coeffs(s2[0], q2[0], g2, be2)

    out = pl.pallas_call(
        _passC_kernel,
        grid=grid,
        in_specs=[
            pl.BlockSpec((_TB, _H), lambda t: (t, 0)),
            pl.BlockSpec((1, _H), lambda t: (0, 0)),
            pl.BlockSpec((1, _H), lambda t: (0, 0)),
            pl.BlockSpec((_H, 1), lambda t: (0, 0)),
            pl.BlockSpec((1, 1), lambda t: (0, 0)),
            pl.BlockSpec((_TB, 1), lambda t: (t, 0)),
        ],
        out_specs=pl.BlockSpec((_TB, 1), lambda t: (t, 0)),
        out_shape=jax.ShapeDtypeStruct((_B, 1), jnp.float32),
    )(h2, a2, c2, W3, b3[None, :], fm)

    return out[:, 0]


# speedup vs baseline: 1.1249x; 1.1249x over previous
import functools

import jax
import jax.numpy as jnp
from jax import lax
from jax.experimental import pallas as pl
from jax.experimental.pallas import tpu as pltpu
from jax.experimental.pallas import tpu_sc as plsc
import numpy as np

_F = 26
_D = 16
_B = 16384
_H = 400
_EIN = _F * _D
_EPAD = 512
_OFFS = np.array((0, *np.cumsum([100000] * _F)[:-1]), dtype=np.int32)

_NW = 32
_CHUNK = 128
_NR = _B // _CHUNK
_RPW = _NR // _NW
_IPC = _CHUNK * _F

_rd = np.arange(_NR)[:, None, None]
_f = np.arange(_F)[None, :, None]
_k = np.arange(_CHUNK)[None, None, :]
_b = _rd * _CHUNK + _k
_OFF16 = (((_b // 8) * 4 + _f // 8) * 64 + (_b % 8) * 8 + _f % 8).astype(np.int32)
del _rd, _f, _k, _b


def _sc_gather_body(tmp_hbm, off_hbm, emb_hbm, lin_hbm,
                    out2_hbm, lsum_hbm,
                    idx_v, off_v, emb_v, lin_v, ls_v,
                    sem_e, sem_l, sem_s):
    nc = 2
    wid = lax.axis_index("s") * nc + lax.axis_index("c")

    def round_body(ci, carry):
        rd = wid * _RPW + ci
        pltpu.sync_copy(tmp_hbm.at[rd], idx_v)
        pltpu.sync_copy(off_hbm.at[rd], off_v)
        ge, gl, sc = [], [], []
        for j in range(_F):
            ge.append(pltpu.async_copy(
                emb_hbm.at[idx_v.at[j]],
                emb_v.at[pl.ds(j * _CHUNK, _CHUNK)], sem_e))
        for j in range(_F):
            gl.append(pltpu.async_copy(
                lin_hbm.at[idx_v.at[j]], lin_v.at[j], sem_l))
        for cp in ge:
            cp.wait()
        for j in range(_F):
            sc.append(pltpu.async_copy(
                emb_v.at[pl.ds(j * _CHUNK, _CHUNK)],
                out2_hbm.at[off_v.at[j]], sem_s))
        for cp in gl:
            cp.wait()
        for ch in range(_CHUNK // 16):
            acc = jnp.zeros((16,), jnp.float32)
            for j in range(_F):
                acc = acc + lin_v[j, pl.ds(ch * 16, 16)]
            ls_v[pl.ds(ch * 16, 16)] = acc
        pltpu.sync_copy(ls_v, lsum_hbm.at[pl.ds(rd * _CHUNK, _CHUNK)])
        for cp in sc:
            cp.wait()
        return carry

    lax.fori_loop(0, _RPW, round_body, 0)


def _sc_gather(tmp3d, off3d, emb_table, lin_flat):
    k = functools.partial(
        pl.kernel,
        out_type=[
            jax.ShapeDtypeStruct((_B * _EPAD // _D, _D), jnp.float32),
            jax.ShapeDtypeStruct((_B,), jnp.float32),
        ],
        mesh=plsc.VectorSubcoreMesh(core_axis_name="c", subcore_axis_name="s"),
        compiler_params=pltpu.CompilerParams(use_tc_tiling_on_sc=False),
        scratch_types=[
            pltpu.VMEM((_F, _CHUNK), jnp.int32),
            pltpu.VMEM((_F, _CHUNK), jnp.int32),
            pltpu.VMEM((_IPC, _D), jnp.float32),
            pltpu.VMEM((_F, _CHUNK), jnp.float32),
            pltpu.VMEM((_CHUNK,), jnp.float32),
            pltpu.SemaphoreType.DMA,
            pltpu.SemaphoreType.DMA,
            pltpu.SemaphoreType.DMA,
        ],
    )(_sc_gather_body)
    return k(tmp3d, off3d, emb_table, lin_flat)


_TB = 512
_NT = _B // _TB


def _mlp_kernel(e4_ref, ls_ref, s_mat_ref, w1_ref, b1_ref, bias_ref,
                g1_ref, be1_ref, w2_ref, b2_ref, g2_ref, be2_ref,
                w3_ref, b3_ref,
                out_ref,
                h_ref, fm_ref, s_ref, q_ref, a_ref, c_ref):
    p = pl.program_id(0)
    t = pl.program_id(1)
    ksz = (128, 128, 128, 32)

    @pl.when(jnp.logical_and(p == 0, t == 0))
    def _():
        s_ref[...] = jnp.zeros_like(s_ref)
        q_ref[...] = jnp.zeros_like(q_ref)

    @pl.when(p == 0)
    def _():
        h1 = jnp.zeros((_TB, _H), jnp.float32)
        es = jnp.zeros((_TB, _D), jnp.float32)
        qs = jnp.zeros((_TB, _D), jnp.float32)
        for c in range(4):
            ec = e4_ref[:, c].reshape(_TB, 128)[:, :ksz[c]]
            w1c = w1_ref[pl.ds(c * 128, ksz[c]), :]
            sc = s_mat_ref[pl.ds(c * 128, ksz[c]), :]
            h1 += jnp.dot(ec, w1c, preferred_element_type=jnp.float32)
            es += jnp.dot(ec, sc, preferred_element_type=jnp.float32)
            qs += jnp.dot(ec * ec, sc, preferred_element_type=jnp.float32)
        h1 = h1 + b1_ref[...]
        h_ref[pl.ds(t * _TB, _TB), :] = h1
        s_ref[...] += jnp.sum(h1, axis=0, keepdims=True)
        q_ref[...] += jnp.sum(h1 * h1, axis=0, keepdims=True)
        inner = 0.5 * jnp.sum(es * es - qs, axis=1, keepdims=True)
        fm_ref[pl.ds(t * _TB, _TB), :] = inner + ls_ref[...] + bias_ref[...]

    @pl.when(jnp.logical_and(p == 1, t == 0))
    def _():
        m = s_ref[...] / _B
        v = q_ref[...] / _B - m * m
        a = g1_ref[...] * lax.rsqrt(v + 1e-5)
        a_ref[...] = a
        c_ref[...] = be1_ref[...] - m * a
        s_ref[...] = jnp.zeros_like(s_ref)
        q_ref[...] = jnp.zeros_like(q_ref)

    @pl.when(p == 1)
    def _():
        h1 = h_ref[pl.ds(t * _TB, _TB), :]
        h = jnp.maximum(h1 * a_ref[...] + c_ref[...], 0.0)
        h2 = jnp.dot(h, w2_ref[...], preferred_element_type=jnp.float32)
        h2 = h2 + b2_ref[...]
        h_ref[pl.ds(t * _TB, _TB), :] = h2
        s_ref[...] += jnp.sum(h2, axis=0, keepdims=True)
        q_ref[...] += jnp.sum(h2 * h2, axis=0, keepdims=True)

    @pl.when(jnp.logical_and(p == 2, t == 0))
    def _():
        m = s_ref[...] / _B
        v = q_ref[...] / _B - m * m
        a = g2_ref[...] * lax.rsqrt(v + 1e-5)
        a_ref[...] = a
        c_ref[...] = be2_ref[...] - m * a

    @pl.when(p == 2)
    def _():
        h2 = h_ref[pl.ds(t * _TB, _TB), :]
        h = jnp.maximum(h2 * a_ref[...] + c_ref[...], 0.0)
        mlp = jnp.dot(h, w3_ref[...], preferred_element_type=jnp.float32)
        out_ref[...] = jax.nn.sigmoid(
            mlp + b3_ref[...] + fm_ref[pl.ds(t * _TB, _TB), :])


def _mlp(e4, ls2, s_mat, W1, b1, bias, g1, be1, W2, b2, g2, be2, W3, b3):
    row = lambda p, t: (jnp.where(p == 0, t, 0), 0)
    zero2 = lambda p, t: (0, 0)
    e4map = lambda p, t: (jnp.where(p == 0, t, 0), 0, 0, 0)
    return pl.pallas_call(
        _mlp_kernel,
        grid=(3, _NT),
        in_specs=[
            pl.BlockSpec((_TB // 8, 4, 8, 128), e4map),
            pl.BlockSpec((_TB, 1), row),
            pl.BlockSpec((_EIN, _D), zero2),
            pl.BlockSpec((_EIN, _H), zero2),
            pl.BlockSpec((1, _H), zero2),
            pl.BlockSpec((1, 1), zero2),
            pl.BlockSpec((1, _H), zero2),
            pl.BlockSpec((1, _H), zero2),
            pl.BlockSpec((_H, _H), zero2),
            pl.BlockSpec((1, _H), zero2),
            pl.BlockSpec((1, _H), zero2),
            pl.BlockSpec((1, _H), zero2),
            pl.BlockSpec((_H, 1), zero2),
            pl.BlockSpec((1, 1), zero2),
        ],
        out_specs=pl.BlockSpec((_TB, 1), lambda p, t: (jnp.where(p == 2, t, 0), 0)),
        out_shape=jax.ShapeDtypeStruct((_B, 1), jnp.float32),
        scratch_shapes=[
            pltpu.VMEM((_B, _H), jnp.float32),
            pltpu.VMEM((_B, 1), jnp.float32),
            pltpu.VMEM((1, _H), jnp.float32),
            pltpu.VMEM((1, _H), jnp.float32),
            pltpu.VMEM((1, _H), jnp.float32),
            pltpu.VMEM((1, _H), jnp.float32),
        ],
    )(e4, ls2, s_mat, W1, b1, bias, g1, be1, W2, b2, g2, be2, W3, b3)


_S_MAT = np.tile(np.eye(_D, dtype=np.float32), (_F, 1))


def kernel(x, emb_table, lin_table, bias, W1, b1, g1, be1,
           W2, b2, g2, be2, W3, b3):
    tmp = x + jnp.asarray(_OFFS, dtype=x.dtype)[None, :]
    tmp3d = tmp.reshape(_NR, _CHUNK, _F).transpose(0, 2, 1)
    off3d = jnp.asarray(_OFF16)

    out2, lsum = _sc_gather(tmp3d, off3d, emb_table, lin_table[:, 0])
    e4 = out2.reshape(_B // 8, 4, 8, 128)
    ls2 = lsum.reshape(_B, 1)

    out = _mlp(e4, ls2, jnp.asarray(_S_MAT), W1, b1[None, :], bias[None, :],
               g1[None, :], be1[None, :], W2, b2[None, :], g2[None, :],
               be2[None, :], W3, b3[None, :])
    return out[:, 0]
